# baseline (speedup 1.0000x reference)
import functools

import jax
import jax.numpy as jnp
from jax import lax
from jax.experimental import pallas as pl
from jax.experimental.pallas import tpu as pltpu
from jax.experimental.pallas import tpu_sc as plsc

NUM_BANDS = 8192
K_BANDS = 2048
ROW_BLOCK = 480

_NC = 2
_NS = 16
_NW = _NC * _NS
_CHUNK = NUM_BANDS // _NS
_OUT_CHUNK = NUM_BANDS // _NW
_LANES = 16


def _sc_mask(w_hbm, mask_hbm, wm_hbm, cnts_hbm, w_v, cnt_v, all_v, outm_v, outwm_v):
    cid = lax.axis_index("c")
    sid = lax.axis_index("s")

    pltpu.sync_copy(w_hbm.at[pl.ds(sid * _CHUNK, _CHUNK)], w_v)

    lanes_i = lax.iota(jnp.int32, _LANES)
    kvec = jnp.full((_LANES,), K_BANDS, jnp.int32)

    def round_body(shift, thr):
        cnt_vec = jnp.zeros((_LANES,), jnp.int32)
        for j in range(_LANES):
            if (j << shift) >= 2**31:
                continue
            cand = thr | (j << shift)
            acc = jnp.zeros((_LANES,), jnp.int32)
            one = jnp.full((_LANES,), 1, jnp.int32)
            zero = jnp.zeros((_LANES,), jnp.int32)
            for c in range(_CHUNK // _LANES):
                v = w_v[pl.ds(c * _LANES, _LANES)]
                b = lax.bitcast_convert_type(v, jnp.int32) & jnp.int32(0x7FFFFFFF)
                acc = acc + jnp.where(b >= cand, one, zero)
            total = jnp.sum(acc)
            cnt_vec = jnp.where(lanes_i == j, total, cnt_vec)
        cnt_v[0] = cnt_vec
        pltpu.sync_copy(cnt_v, cnts_hbm.at[pl.ds(sid, 1)])
        plsc.subcore_barrier()
        pltpu.sync_copy(cnts_hbm, all_v)
        gcnt = jnp.zeros((_LANES,), jnp.int32)
        for s in range(_NS):
            gcnt = gcnt + all_v[s]
        plsc.subcore_barrier()
        m = jnp.sum(jnp.where(gcnt >= kvec,
                              jnp.full((_LANES,), 1, jnp.int32),
                              jnp.zeros((_LANES,), jnp.int32))) - 1
        return thr | lax.shift_left(m, shift)

    thr = jnp.int32(0)
    for shift in range(28, -1, -4):
        thr = round_body(shift, thr)

    off = cid * _OUT_CHUNK
    for c in range(_OUT_CHUNK // _LANES):
        v = w_v[pl.ds(off + c * _LANES, _LANES)]
        b = lax.bitcast_convert_type(v, jnp.int32) & jnp.int32(0x7FFFFFFF)
        m_f = jnp.where(b >= thr,
                        jnp.full((_LANES,), 1.0, jnp.float32),
                        jnp.zeros((_LANES,), jnp.float32))
        outm_v[pl.ds(c * _LANES, _LANES)] = m_f
        outwm_v[pl.ds(c * _LANES, _LANES)] = v * m_f
    base = sid * _CHUNK + cid * _OUT_CHUNK
    pltpu.sync_copy(outm_v, mask_hbm.at[pl.ds(base, _OUT_CHUNK)])
    pltpu.sync_copy(outwm_v, wm_hbm.at[pl.ds(base, _OUT_CHUNK)])


_sc_mask_call = functools.partial(
    pl.kernel,
    out_type=[
        jax.ShapeDtypeStruct((NUM_BANDS,), jnp.float32),
        jax.ShapeDtypeStruct((NUM_BANDS,), jnp.float32),
        jax.ShapeDtypeStruct((_NS, _LANES), jnp.int32),
    ],
    mesh=plsc.VectorSubcoreMesh(core_axis_name="c", subcore_axis_name="s"),
    compiler_params=pltpu.CompilerParams(needs_layout_passes=False),
    scratch_types=[
        pltpu.VMEM((_CHUNK,), jnp.float32),
        pltpu.VMEM((1, _LANES), jnp.int32),
        pltpu.VMEM((_NS, _LANES), jnp.int32),
        pltpu.VMEM((_OUT_CHUNK,), jnp.float32),
        pltpu.VMEM((_OUT_CHUNK,), jnp.float32),
    ],
)(_sc_mask)


def _affine_kernel(x_ref, wm_ref, b_ref, out_ref):
    out_ref[...] = x_ref[...] * wm_ref[...] + b_ref[...]


@jax.jit
def kernel(x, weights, bias):
    batch, num_bands = x.shape
    mask, wm, _ = _sc_mask_call(weights)
    wm2 = wm.reshape(1, num_bands)
    b2 = bias.reshape(1, num_bands)
    grid = (pl.cdiv(batch, ROW_BLOCK),)
    out = pl.pallas_call(
        _affine_kernel,
        grid=grid,
        in_specs=[
            pl.BlockSpec((ROW_BLOCK, num_bands), lambda i: (i, 0)),
            pl.BlockSpec((1, num_bands), lambda i: (0, 0)),
            pl.BlockSpec((1, num_bands), lambda i: (0, 0)),
        ],
        out_specs=pl.BlockSpec((ROW_BLOCK, num_bands), lambda i: (i, 0)),
        out_shape=jax.ShapeDtypeStruct((batch, num_bands), jnp.float32),
        compiler_params=pltpu.CompilerParams(
            vmem_limit_bytes=128 * 1024 * 1024,
        ),
    )(x, wm2, b2)
    return out, mask


# speedup vs baseline: 1.4782x; 1.4782x over previous
import functools

import jax
import jax.numpy as jnp
from jax.experimental import pallas as pl
from jax.experimental.pallas import tpu as pltpu

NUM_BANDS = 8192
K_BANDS = 2048
ROW_BLOCK = 480


def _gating_kernel(x_ref, w_ref, b_ref, out_ref, mask_out_ref, mask_scr):
    @pl.when(pl.program_id(0) == 0)
    def _compute_mask():
        w = w_ref[...]
        bits = jax.lax.bitcast_convert_type(w, jnp.int32) & jnp.int32(0x7FFFFFFF)
        thr = jnp.int32(0)
        for bit in range(30, -1, -1):
            cand = thr | jnp.int32(1 << bit)
            cnt = jnp.sum((bits >= cand).astype(jnp.int32))
            thr = jnp.where(cnt >= K_BANDS, cand, thr)
        mask = (bits >= thr).astype(jnp.float32)
        mask_out_ref[...] = mask
        mask_scr[...] = w * mask

    out_ref[...] = x_ref[...] * mask_scr[...] + b_ref[...]


@jax.jit
def kernel(x, weights, bias):
    batch, num_bands = x.shape
    w2 = weights.reshape(1, num_bands)
    b2 = bias.reshape(1, num_bands)
    grid = (pl.cdiv(batch, ROW_BLOCK),)
    out, mask = pl.pallas_call(
        _gating_kernel,
        grid=grid,
        in_specs=[
            pl.BlockSpec((ROW_BLOCK, num_bands), lambda i: (i, 0)),
            pl.BlockSpec((1, num_bands), lambda i: (0, 0)),
            pl.BlockSpec((1, num_bands), lambda i: (0, 0)),
        ],
        out_specs=[
            pl.BlockSpec((ROW_BLOCK, num_bands), lambda i: (i, 0)),
            pl.BlockSpec((1, num_bands), lambda i: (0, 0)),
        ],
        out_shape=[
            jax.ShapeDtypeStruct((batch, num_bands), jnp.float32),
            jax.ShapeDtypeStruct((1, num_bands), jnp.float32),
        ],
        scratch_shapes=[pltpu.VMEM((1, num_bands), jnp.float32)],
        compiler_params=pltpu.CompilerParams(
            vmem_limit_bytes=128 * 1024 * 1024,
        ),
    )(x, w2, b2)
    return out, mask.reshape(num_bands)


# speedup vs baseline: 1.4800x; 1.0012x over previous
import jax
import jax.numpy as jnp
from jax.experimental import pallas as pl
from jax.experimental.pallas import tpu as pltpu

NUM_BANDS = 8192
K_BANDS = 2048
ROW_BLOCK = 480


def _gating_kernel(x_ref, w_ref, b_ref, out_ref, mask_out_ref, mask_scr):
    @pl.when(pl.program_id(0) == 0)
    def _compute_mask():
        w = w_ref[...]
        bits = jax.lax.bitcast_convert_type(w, jnp.int32) & jnp.int32(0x7FFFFFFF)
        thr = jnp.int32(0)
        for bit in range(30, -1, -1):
            cand = thr | jnp.int32(1 << bit)
            cnt = jnp.sum((bits >= cand).astype(jnp.int32))
            thr = jnp.where(cnt >= K_BANDS, cand, thr)
        mask = (bits >= thr).astype(jnp.float32)
        mask_out_ref[...] = mask
        mask_scr[...] = w * mask

    out_ref[...] = x_ref[...] * mask_scr[...] + b_ref[...]


@jax.jit
def kernel(x, weights, bias):
    batch, num_bands = x.shape
    w2 = weights.reshape(1, num_bands)
    b2 = bias.reshape(1, num_bands)
    grid = (pl.cdiv(batch, ROW_BLOCK),)
    out, mask = pl.pallas_call(
        _gating_kernel,
        grid=grid,
        in_specs=[
            pl.BlockSpec((ROW_BLOCK, num_bands), lambda i: (i, 0)),
            pl.BlockSpec((1, num_bands), lambda i: (0, 0)),
            pl.BlockSpec((1, num_bands), lambda i: (0, 0)),
        ],
        out_specs=[
            pl.BlockSpec((ROW_BLOCK, num_bands), lambda i: (i, 0)),
            pl.BlockSpec((1, num_bands), lambda i: (0, 0)),
        ],
        out_shape=[
            jax.ShapeDtypeStruct((batch, num_bands), jnp.float32),
            jax.ShapeDtypeStruct((1, num_bands), jnp.float32),
        ],
        scratch_shapes=[pltpu.VMEM((1, num_bands), jnp.float32)],
        compiler_params=pltpu.CompilerParams(
            vmem_limit_bytes=128 * 1024 * 1024,
        ),
    )(x, w2, b2)
    return out, mask.reshape(num_bands)
